# baseline (speedup 1.0000x reference)
import functools

import jax
import jax.numpy as jnp
from jax import lax
from jax.experimental import pallas as pl
from jax.experimental.pallas import tpu as pltpu
from jax.experimental.pallas import tpu_sc as plsc

T = 2048
D = 1024
F = 1024
E = 8
SOFTCAP = 30.0

BT = 256
NB = 24
S = NB * BT

NW = 32
DISPATCH_CHUNK = 32
DISPATCH_NBUF = 3
ROWS_PER_W = S // NW
DISPATCH_NCH = ROWS_PER_W // DISPATCH_CHUNK
COMB_CHUNK = 16
TOK_PER_W = T // NW
COMB_NCH = TOK_PER_W // COMB_CHUNK

_LANES = 128
_NEG = -1e30


def _cumsum_shift(a, axis):
    n = a.shape[axis]
    sh = 1
    while sh < n:
        if axis == 0:
            shifted = jnp.concatenate(
                [jnp.zeros((sh, a.shape[1]), a.dtype), a[:-sh]], axis=0)
        else:
            shifted = jnp.concatenate(
                [jnp.zeros((a.shape[0], sh), a.dtype), a[:, :-sh]], axis=1)
        a = a + shifted
        sh *= 2
    return a


def _router_body(x_ref, gw_ref, d1_ref, d2_ref, w1_ref, w2_ref, be_ref):
    x = x_ref[...]
    gw = gw_ref[...]
    logits = jnp.dot(x, gw, preferred_element_type=jnp.float32)
    l = jnp.tanh(logits / SOFTCAP)
    lane = lax.broadcasted_iota(jnp.int32, l.shape, 1)
    valid = lane < E
    l = jnp.where(valid, l, _NEG)
    m1 = jnp.max(l, axis=1, keepdims=True)
    i1 = jnp.min(jnp.where(l == m1, lane, _LANES), axis=1, keepdims=True)
    l2 = jnp.where(lane == i1, _NEG, l)
    m2 = jnp.max(l2, axis=1, keepdims=True)
    i2 = jnp.min(jnp.where(l2 == m2, lane, _LANES), axis=1, keepdims=True)
    w1 = 1.0 / (1.0 + jnp.exp(m2 - m1))
    w2 = 1.0 - w1

    oh1 = jnp.where(lane == i1, 1, 0)
    oh2 = jnp.where(lane == i2, 1, 0)
    oh = oh1 + oh2
    cum = _cumsum_shift(oh, axis=0)
    excl = cum - oh
    counts = cum[T - 1:T, :]
    nblk = (counts + (BT - 1)) // BT
    bcum = _cumsum_shift(nblk, axis=1)
    pad_off = (bcum - nblk) * BT
    rank1 = jnp.sum(oh1 * excl, axis=1, keepdims=True)
    rank2 = jnp.sum(oh2 * excl, axis=1, keepdims=True)
    off1 = jnp.sum(oh1 * pad_off, axis=1, keepdims=True)
    off2 = jnp.sum(oh2 * pad_off, axis=1, keepdims=True)
    d1 = rank1 + off1
    d2 = rank2 + off2

    shp = l.shape
    d1_ref[...] = jnp.broadcast_to(d1, shp)
    d2_ref[...] = jnp.broadcast_to(d2, shp)
    w1_ref[...] = jnp.broadcast_to(w1, shp)
    w2_ref[...] = jnp.broadcast_to(w2, shp)

    brow = lax.broadcasted_iota(jnp.int32, (32, _LANES), 0)
    blane = lax.broadcasted_iota(jnp.int32, (32, _LANES), 1)
    ge = jnp.where((brow >= jnp.broadcast_to(bcum, (32, _LANES))) & (blane < E), 1, 0)
    bexp = jnp.minimum(jnp.sum(ge, axis=1, keepdims=True), E - 1)
    be_ref[...] = jnp.broadcast_to(bexp, (32, _LANES))


def _router(x, gate_w):
    gw_pad = jnp.zeros((D, _LANES), jnp.float32).at[:, :E].set(gate_w)
    d1, d2, w1, w2, be = pl.pallas_call(
        _router_body,
        out_shape=[
            jax.ShapeDtypeStruct((T, _LANES), jnp.int32),
            jax.ShapeDtypeStruct((T, _LANES), jnp.int32),
            jax.ShapeDtypeStruct((T, _LANES), jnp.float32),
            jax.ShapeDtypeStruct((T, _LANES), jnp.float32),
            jax.ShapeDtypeStruct((32, _LANES), jnp.int32),
        ],
    )(x, gw_pad)
    return d1[:, 0], d2[:, 0], w1[:, 0], w2[:, 0], be[:NB, 0]


def _combine_weights(s0, s1, w1, w2):
    w_d = jnp.zeros(S, jnp.float32).at[s0].set(w1).at[s1].set(w2)
    return jnp.broadcast_to(w_d[:, None], (S, _LANES))


def _sc_mesh():
    return plsc.VectorSubcoreMesh(core_axis_name="c", subcore_axis_name="s")


def _dispatch_body(x_hbm, d1_hbm, d2_hbm, xd_hbm, idx_v, rows_v, s1m, s2m):
    wid = lax.axis_index("s") * 2 + lax.axis_index("c")
    base = wid * TOK_PER_W
    pltpu.sync_copy(d1_hbm.at[wid], idx_v.at[0])
    pltpu.sync_copy(d2_hbm.at[wid], idx_v.at[1])
    pltpu.sync_copy(x_hbm.at[pl.ds(base, TOK_PER_W)], rows_v)
    cp1 = pltpu.async_copy(rows_v, xd_hbm.at[idx_v.at[0]], s1m)
    cp2 = pltpu.async_copy(rows_v, xd_hbm.at[idx_v.at[1]], s2m)
    cp1.wait()
    cp2.wait()


def _dispatch_scatter(xb, s0, s1):
    k = functools.partial(
        pl.kernel,
        out_type=jax.ShapeDtypeStruct((S, D // 2), jnp.int32),
        mesh=_sc_mesh(),
        scratch_types=[
            pltpu.VMEM((2, TOK_PER_W), jnp.int32),
            pltpu.VMEM((TOK_PER_W, D // 2), jnp.int32),
            pltpu.SemaphoreType.DMA,
            pltpu.SemaphoreType.DMA,
        ],
    )(_dispatch_body)
    return k(xb, s0.reshape(NW, TOK_PER_W), s1.reshape(NW, TOK_PER_W))


def _gmm_body(be_ref, x_ref, win_ref, wout_ref, ws_ref, y_ref):
    x = x_ref[...].astype(jnp.float32)
    h = jnp.dot(x, win_ref[0], preferred_element_type=jnp.float32)
    h = jax.nn.gelu(h)
    y = jnp.dot(h, wout_ref[0], preferred_element_type=jnp.float32)
    y_ref[...] = y * ws_ref[...][:, 0:1]


def _gmm(x_d, w_in, w_out, w_bcast, block_expert):
    grid_spec = pltpu.PrefetchScalarGridSpec(
        num_scalar_prefetch=1,
        grid=(NB,),
        in_specs=[
            pl.BlockSpec((BT, D), lambda b, be: (b, 0)),
            pl.BlockSpec((1, D, F), lambda b, be: (be[b], 0, 0)),
            pl.BlockSpec((1, F, D), lambda b, be: (be[b], 0, 0)),
            pl.BlockSpec((BT, _LANES), lambda b, be: (b, 0)),
        ],
        out_specs=pl.BlockSpec((BT, D), lambda b, be: (b, 0)),
    )
    return pl.pallas_call(
        _gmm_body,
        grid_spec=grid_spec,
        out_shape=jax.ShapeDtypeStruct((S, D), jnp.float32),
        compiler_params=pltpu.CompilerParams(
            dimension_semantics=("arbitrary",),
        ),
    )(block_expert, x_d, w_in, w_out, w_bcast)


def _combine_body(y_hbm, s0_hbm, s1_hbm, out_hbm, i0_v, i1_v,
                  r0a, r0b, r1a, r1b, g0a, g0b, g1a, g1b, wa, wb):
    r0 = (r0a, r0b)
    r1 = (r1a, r1b)
    g0sem = (g0a, g0b)
    g1sem = (g1a, g1b)
    wsem = (wa, wb)
    wid = lax.axis_index("s") * 2 + lax.axis_index("c")
    base = wid * TOK_PER_W
    pltpu.sync_copy(s0_hbm.at[wid], i0_v)
    pltpu.sync_copy(s1_hbm.at[wid], i1_v)
    g0cp, g1cp, wcp = {}, {}, {}

    def start_gathers(c):
        b = c & 1
        g0cp[c] = pltpu.async_copy(y_hbm.at[i0_v.at[c]], r0[b], g0sem[b])
        g1cp[c] = pltpu.async_copy(y_hbm.at[i1_v.at[c]], r1[b], g1sem[b])

    start_gathers(0)
    for c in range(COMB_NCH):
        b = c & 1
        g0cp[c].wait()
        g1cp[c].wait()
        if c + 1 < COMB_NCH:
            if c - 1 >= 0:
                wcp[c - 1].wait()
            start_gathers(c + 1)

        def add_body(j, _):
            for i in range(COMB_CHUNK):
                sl = pl.ds(j * 16, 16)
                r0[b][i, sl] = r0[b][i, sl] + r1[b][i, sl]
            return 0

        lax.fori_loop(0, D // 16, add_body, 0)
        wcp[c] = pltpu.async_copy(
            r0[b], out_hbm.at[pl.ds(base + c * COMB_CHUNK, COMB_CHUNK)], wsem[b])
    for c in range(max(0, COMB_NCH - 2), COMB_NCH):
        wcp[c].wait()


def _combine(y_d, s0, s1):
    k = functools.partial(
        pl.kernel,
        out_type=jax.ShapeDtypeStruct((T, D), jnp.float32),
        mesh=_sc_mesh(),
        scratch_types=[
            pltpu.VMEM((COMB_NCH, COMB_CHUNK), jnp.int32),
            pltpu.VMEM((COMB_NCH, COMB_CHUNK), jnp.int32),
            pltpu.VMEM((COMB_CHUNK, D), jnp.float32),
            pltpu.VMEM((COMB_CHUNK, D), jnp.float32),
            pltpu.VMEM((COMB_CHUNK, D), jnp.float32),
            pltpu.VMEM((COMB_CHUNK, D), jnp.float32),
            pltpu.SemaphoreType.DMA,
            pltpu.SemaphoreType.DMA,
            pltpu.SemaphoreType.DMA,
            pltpu.SemaphoreType.DMA,
            pltpu.SemaphoreType.DMA,
            pltpu.SemaphoreType.DMA,
        ],
    )(_combine_body)
    return k(y_d, s0.reshape(NW, COMB_NCH, COMB_CHUNK), s1.reshape(NW, COMB_NCH, COMB_CHUNK))


def kernel(hidden_states, gate_w, w_in, w_out):
    x = hidden_states.astype(jnp.float32)
    s0, s1, w1, w2, block_expert = _router(x, gate_w)
    xb_packed = lax.bitcast_convert_type(
        x.astype(jnp.bfloat16).reshape(T, D // 2, 2), jnp.int32)
    xd_packed = _dispatch_scatter(xb_packed, s0, s1)
    x_d = lax.bitcast_convert_type(
        xd_packed[..., None], jnp.bfloat16).reshape(S, D)
    w_bcast = _combine_weights(s0, s1, w1, w2)
    y_d = _gmm(x_d, w_in, w_out, w_bcast, block_expert)
    return _combine(y_d, s0, s1)


# speedup vs baseline: 2.3434x; 2.3434x over previous
import functools

import jax
import jax.numpy as jnp
from jax import lax
from jax.experimental import pallas as pl
from jax.experimental.pallas import tpu as pltpu
from jax.experimental.pallas import tpu_sc as plsc

T = 2048
D = 1024
F = 1024
E = 8
SOFTCAP = 30.0

BT = 256
NB = 24
S = NB * BT

NW = 32
DISPATCH_CHUNK = 32
DISPATCH_NBUF = 3
ROWS_PER_W = S // NW
DISPATCH_NCH = ROWS_PER_W // DISPATCH_CHUNK
COMB_CHUNK = 16
TOK_PER_W = T // NW
COMB_NCH = TOK_PER_W // COMB_CHUNK

_LANES = 128
_NEG = -1e30


def _cumsum_shift(a, axis):
    n = a.shape[axis]
    sh = 1
    while sh < n:
        if axis == 0:
            shifted = jnp.concatenate(
                [jnp.zeros((sh, a.shape[1]), a.dtype), a[:-sh]], axis=0)
        else:
            shifted = jnp.concatenate(
                [jnp.zeros((a.shape[0], sh), a.dtype), a[:, :-sh]], axis=1)
        a = a + shifted
        sh *= 2
    return a


def _router_body(x_ref, gw_ref, d1_ref, d2_ref, w1_ref, w2_ref, be_ref):
    x = x_ref[...]
    gw = gw_ref[...]
    logits = jnp.dot(x, gw, preferred_element_type=jnp.float32)
    l = jnp.tanh(logits / SOFTCAP)
    lane = lax.broadcasted_iota(jnp.int32, l.shape, 1)
    valid = lane < E
    l = jnp.where(valid, l, _NEG)
    m1 = jnp.max(l, axis=1, keepdims=True)
    i1 = jnp.min(jnp.where(l == m1, lane, _LANES), axis=1, keepdims=True)
    l2 = jnp.where(lane == i1, _NEG, l)
    m2 = jnp.max(l2, axis=1, keepdims=True)
    i2 = jnp.min(jnp.where(l2 == m2, lane, _LANES), axis=1, keepdims=True)
    w1 = 1.0 / (1.0 + jnp.exp(m2 - m1))
    w2 = 1.0 - w1

    oh1 = jnp.where(lane == i1, 1, 0)
    oh2 = jnp.where(lane == i2, 1, 0)
    oh = oh1 + oh2
    cum = _cumsum_shift(oh, axis=0)
    excl = cum - oh
    counts = cum[T - 1:T, :]
    nblk = (counts + (BT - 1)) // BT
    bcum = _cumsum_shift(nblk, axis=1)
    pad_off = (bcum - nblk) * BT
    rank1 = jnp.sum(oh1 * excl, axis=1, keepdims=True)
    rank2 = jnp.sum(oh2 * excl, axis=1, keepdims=True)
    off1 = jnp.sum(oh1 * pad_off, axis=1, keepdims=True)
    off2 = jnp.sum(oh2 * pad_off, axis=1, keepdims=True)
    d1 = rank1 + off1
    d2 = rank2 + off2

    shp = l.shape
    d1_ref[...] = jnp.broadcast_to(d1, shp)
    d2_ref[...] = jnp.broadcast_to(d2, shp)
    w1_ref[...] = jnp.broadcast_to(w1, shp)
    w2_ref[...] = jnp.broadcast_to(w2, shp)

    brow = lax.broadcasted_iota(jnp.int32, (32, _LANES), 0)
    blane = lax.broadcasted_iota(jnp.int32, (32, _LANES), 1)
    ge = jnp.where((brow >= jnp.broadcast_to(bcum, (32, _LANES))) & (blane < E), 1, 0)
    bexp = jnp.sum(ge, axis=1, keepdims=True)
    be_ref[...] = jnp.broadcast_to(bexp, (32, _LANES))


def _router(x, gate_w):
    gw_pad = jnp.zeros((D, _LANES), jnp.float32).at[:, :E].set(gate_w)
    d1, d2, w1, w2, be = pl.pallas_call(
        _router_body,
        out_shape=[
            jax.ShapeDtypeStruct((T, _LANES), jnp.int32),
            jax.ShapeDtypeStruct((T, _LANES), jnp.int32),
            jax.ShapeDtypeStruct((T, _LANES), jnp.float32),
            jax.ShapeDtypeStruct((T, _LANES), jnp.float32),
            jax.ShapeDtypeStruct((32, _LANES), jnp.int32),
        ],
    )(x, gw_pad)
    return d1[:, 0], d2[:, 0], w1[:, 0], w2[:, 0], be[:NB, 0]


def _combine_weights(s0, s1, w1, w2):
    w_d = jnp.zeros(S, jnp.float32).at[s0].set(w1).at[s1].set(w2)
    return jnp.broadcast_to(w_d[:, None], (S, _LANES))


def _sc_mesh():
    return plsc.VectorSubcoreMesh(core_axis_name="c", subcore_axis_name="s")


def _dispatch_body(x_hbm, d1_hbm, d2_hbm, xd_hbm, idx_v, rows_v, s1m, s2m):
    wid = lax.axis_index("s") * 2 + lax.axis_index("c")
    base = wid * TOK_PER_W
    cp0 = pltpu.async_copy(x_hbm.at[pl.ds(base, TOK_PER_W)], rows_v, s1m)
    pltpu.sync_copy(d1_hbm.at[wid], idx_v.at[0])
    pltpu.sync_copy(d2_hbm.at[wid], idx_v.at[1])
    cp0.wait()
    cp1 = pltpu.async_copy(rows_v, xd_hbm.at[idx_v.at[0]], s1m)
    cp2 = pltpu.async_copy(rows_v, xd_hbm.at[idx_v.at[1]], s2m)
    cp1.wait()
    cp2.wait()


def _dispatch_scatter(xb, s0, s1):
    k = functools.partial(
        pl.kernel,
        out_type=jax.ShapeDtypeStruct((S, D), jnp.float32),
        mesh=_sc_mesh(),
        scratch_types=[
            pltpu.VMEM((2, TOK_PER_W), jnp.int32),
            pltpu.VMEM((TOK_PER_W, D), jnp.float32),
            pltpu.SemaphoreType.DMA,
            pltpu.SemaphoreType.DMA,
        ],
    )(_dispatch_body)
    return k(xb, s0.reshape(NW, TOK_PER_W), s1.reshape(NW, TOK_PER_W))


def _gmm_body(be_ref, x_ref, win_ref, wout_ref, ws_ref, y_ref):
    @pl.when(be_ref[pl.program_id(0)] < E)
    def _():
        x = x_ref[...]
        h = jnp.dot(x, win_ref[0], preferred_element_type=jnp.float32)
        h = jax.nn.gelu(h)
        y = jnp.dot(h, wout_ref[0], preferred_element_type=jnp.float32)
        y_ref[...] = y * ws_ref[...][:, 0:1]


def _gmm(x_d, w_in, w_out, w_bcast, block_expert):
    grid_spec = pltpu.PrefetchScalarGridSpec(
        num_scalar_prefetch=1,
        grid=(NB,),
        in_specs=[
            pl.BlockSpec((BT, D), lambda b, be: (b, 0)),
            pl.BlockSpec((1, D, F), lambda b, be: (jnp.minimum(be[b], E - 1), 0, 0)),
            pl.BlockSpec((1, F, D), lambda b, be: (jnp.minimum(be[b], E - 1), 0, 0)),
            pl.BlockSpec((BT, _LANES), lambda b, be: (b, 0)),
        ],
        out_specs=pl.BlockSpec((BT, D), lambda b, be: (b, 0)),
    )
    return pl.pallas_call(
        _gmm_body,
        grid_spec=grid_spec,
        out_shape=jax.ShapeDtypeStruct((S, D), jnp.float32),
        compiler_params=pltpu.CompilerParams(
            dimension_semantics=("arbitrary",),
        ),
    )(block_expert, x_d, w_in, w_out, w_bcast)


def _combine_body(y_hbm, s0_hbm, s1_hbm, out_hbm, i0_v, i1_v,
                  r0a, r0b, r1a, r1b, g0a, g0b, g1a, g1b, wa, wb):
    r0 = (r0a, r0b)
    r1 = (r1a, r1b)
    g0sem = (g0a, g0b)
    g1sem = (g1a, g1b)
    wsem = (wa, wb)
    wid = lax.axis_index("s") * 2 + lax.axis_index("c")
    base = wid * TOK_PER_W
    pltpu.sync_copy(s0_hbm.at[wid], i0_v)
    pltpu.sync_copy(s1_hbm.at[wid], i1_v)
    g0cp, g1cp, wcp = {}, {}, {}

    def start_gathers(c):
        b = c & 1
        g0cp[c] = pltpu.async_copy(y_hbm.at[i0_v.at[c]], r0[b], g0sem[b])
        g1cp[c] = pltpu.async_copy(y_hbm.at[i1_v.at[c]], r1[b], g1sem[b])

    start_gathers(0)
    for c in range(COMB_NCH):
        b = c & 1
        g0cp[c].wait()
        g1cp[c].wait()
        if c + 1 < COMB_NCH:
            if c - 1 >= 0:
                wcp[c - 1].wait()
            start_gathers(c + 1)

        def add_body(j, _):
            for i in range(COMB_CHUNK):
                sl = pl.ds(j * 16, 16)
                r0[b][i, sl] = r0[b][i, sl] + r1[b][i, sl]
            return 0

        lax.fori_loop(0, D // 16, add_body, 0)
        wcp[c] = pltpu.async_copy(
            r0[b], out_hbm.at[pl.ds(base + c * COMB_CHUNK, COMB_CHUNK)], wsem[b])
    for c in range(max(0, COMB_NCH - 2), COMB_NCH):
        wcp[c].wait()


def _combine(y_d, s0, s1):
    k = functools.partial(
        pl.kernel,
        out_type=jax.ShapeDtypeStruct((T, D), jnp.float32),
        mesh=_sc_mesh(),
        scratch_types=[
            pltpu.VMEM((COMB_NCH, COMB_CHUNK), jnp.int32),
            pltpu.VMEM((COMB_NCH, COMB_CHUNK), jnp.int32),
            pltpu.VMEM((COMB_CHUNK, D), jnp.float32),
            pltpu.VMEM((COMB_CHUNK, D), jnp.float32),
            pltpu.VMEM((COMB_CHUNK, D), jnp.float32),
            pltpu.VMEM((COMB_CHUNK, D), jnp.float32),
            pltpu.SemaphoreType.DMA,
            pltpu.SemaphoreType.DMA,
            pltpu.SemaphoreType.DMA,
            pltpu.SemaphoreType.DMA,
            pltpu.SemaphoreType.DMA,
            pltpu.SemaphoreType.DMA,
        ],
    )(_combine_body)
    return k(y_d, s0.reshape(NW, COMB_NCH, COMB_CHUNK), s1.reshape(NW, COMB_NCH, COMB_CHUNK))


def kernel(hidden_states, gate_w, w_in, w_out):
    x = hidden_states.astype(jnp.float32)
    s0, s1, w1, w2, block_expert = _router(x, gate_w)
    x_d = _dispatch_scatter(x, s0, s1)
    w_bcast = _combine_weights(s0, s1, w1, w2)
    y_d = _gmm(x_d, w_in, w_out, w_bcast, block_expert)
    return _combine(y_d, s0, s1)
